# Initial kernel scaffold; baseline (speedup 1.0000x reference)
import jax
import jax.numpy as jnp
from jax.experimental import pallas as pl


def kernel(x, gate_W, gate_b, W1, b1, W2, b2):
    raise NotImplementedError("write your pallas kernel here")


# speedup vs baseline: 1.1728x; 1.1728x over previous
import functools

import jax
import jax.numpy as jnp
from jax.experimental import pallas as pl
from jax.experimental.pallas import tpu as pltpu

T = 2048
D = 1024
C = 1024
E = 8
K = 2

BT = 256
EP = 128


def _moe_dense_kernel(x_ref, gw_ref, gb_ref, w1_ref, b1_ref, w2_ref, b2_ref,
                      out_ref, acc_ref, cw_ref):
    e = pl.program_id(1)
    xb = x_ref[...]

    @pl.when(e == 0)
    def _router():
        logits = jax.lax.dot_general(
            xb, gw_ref[...], (((1,), (0,)), ((), ())),
            preferred_element_type=jnp.float32)
        logits = logits + gb_ref[0:1, :]
        lane = jax.lax.broadcasted_iota(jnp.int32, (BT, EP), 1)
        logits = jnp.where(lane < E, logits, -jnp.inf)
        m = jnp.max(logits, axis=-1, keepdims=True)
        p = jnp.exp(logits - m)
        p = jnp.where(lane < E, p, 0.0)
        probs = p / jnp.sum(p, axis=-1, keepdims=True)
        m1 = jnp.max(probs, axis=-1, keepdims=True)
        i1 = jnp.argmax(probs, axis=-1)[:, None]
        probs2 = jnp.where(lane == i1, -1.0, probs)
        m2 = jnp.max(probs2, axis=-1, keepdims=True)
        i2 = jnp.argmax(probs2, axis=-1)[:, None]
        denom = m1 + m2
        cw_ref[...] = (jnp.where(lane == i1, m1 / denom, 0.0)
                       + jnp.where(lane == i2, m2 / denom, 0.0))
        acc_ref[...] = jnp.zeros_like(acc_ref)

    xb16 = xb.astype(jnp.bfloat16)
    h = jax.lax.dot_general(
        xb16, w1_ref[0], (((1,), (0,)), ((), ())),
        preferred_element_type=jnp.float32)
    h = jnp.maximum(h + b1_ref[0], 0.0).astype(jnp.bfloat16)
    y = jax.lax.dot_general(
        h, w2_ref[0], (((1,), (0,)), ((), ())),
        preferred_element_type=jnp.float32)
    y = y + b2_ref[0]

    lane = jax.lax.broadcasted_iota(jnp.int32, (BT, EP), 1)
    w = jnp.sum(jnp.where(lane == e, cw_ref[...], 0.0), axis=-1, keepdims=True)
    acc_ref[...] += w * y

    @pl.when(e == E - 1)
    def _flush():
        out_ref[...] = acc_ref[...]


@functools.partial(jax.jit)
def kernel(x, gate_W, gate_b, W1, b1, W2, b2):
    gw_pad = jnp.zeros((D, EP), jnp.float32).at[:, :E].set(gate_W)
    gb_pad = jnp.zeros((8, EP), jnp.float32).at[:, :E].set(gate_b[None, :])
    w1_16 = W1.astype(jnp.bfloat16)
    w2_16 = W2.astype(jnp.bfloat16)
    b1r = b1.reshape(E, 1, D)
    b2r = b2.reshape(E, 1, C)

    out = pl.pallas_call(
        _moe_dense_kernel,
        grid=(T // BT, E),
        in_specs=[
            pl.BlockSpec((BT, D), lambda t, e: (t, 0)),
            pl.BlockSpec((D, EP), lambda t, e: (0, 0)),
            pl.BlockSpec((8, EP), lambda t, e: (0, 0)),
            pl.BlockSpec((1, D, D), lambda t, e: (e, 0, 0)),
            pl.BlockSpec((1, 1, D), lambda t, e: (e, 0, 0)),
            pl.BlockSpec((1, D, C), lambda t, e: (e, 0, 0)),
            pl.BlockSpec((1, 1, C), lambda t, e: (e, 0, 0)),
        ],
        out_specs=pl.BlockSpec((BT, C), lambda t, e: (t, 0)),
        out_shape=jax.ShapeDtypeStruct((T, C), jnp.float32),
        scratch_shapes=[
            pltpu.VMEM((BT, C), jnp.float32),
            pltpu.VMEM((BT, EP), jnp.float32),
        ],
    )(x, gw_pad, gb_pad, w1_16, b1r, w2_16, b2r)
    return out


# speedup vs baseline: 1.6001x; 1.3643x over previous
import functools

import jax
import jax.numpy as jnp
from jax.experimental import pallas as pl
from jax.experimental.pallas import tpu as pltpu

T = 2048
D = 1024
C = 1024
E = 8
K = 2

BT = 256
EP = 128


def _moe_dense_kernel(x_ref, gw_ref, gb_ref, w1_ref, b1_ref, w2_ref, b2_ref,
                      out_ref):
    xb = x_ref[...]

    logits = jax.lax.dot_general(
        xb, gw_ref[...], (((1,), (0,)), ((), ())),
        preferred_element_type=jnp.float32)
    logits = logits + gb_ref[0:1, :]
    lane = jax.lax.broadcasted_iota(jnp.int32, (BT, EP), 1)
    logits = jnp.where(lane < E, logits, -jnp.inf)
    m = jnp.max(logits, axis=-1, keepdims=True)
    p = jnp.exp(logits - m)
    p = jnp.where(lane < E, p, 0.0)
    probs = p / jnp.sum(p, axis=-1, keepdims=True)
    m1 = jnp.max(probs, axis=-1, keepdims=True)
    i1 = jnp.argmax(probs, axis=-1)[:, None]
    probs2 = jnp.where(lane == i1, -1.0, probs)
    m2 = jnp.max(probs2, axis=-1, keepdims=True)
    i2 = jnp.argmax(probs2, axis=-1)[:, None]
    denom = m1 + m2
    cw = (jnp.where(lane == i1, m1 / denom, 0.0)
          + jnp.where(lane == i2, m2 / denom, 0.0))

    xb16 = xb.astype(jnp.bfloat16)
    acc = jnp.zeros((BT, C), jnp.float32)
    for e in range(E):
        h = jax.lax.dot_general(
            xb16, w1_ref[e], (((1,), (0,)), ((), ())),
            preferred_element_type=jnp.float32)
        h = jnp.maximum(h + b1_ref[e], 0.0).astype(jnp.bfloat16)
        y = jax.lax.dot_general(
            h, w2_ref[e], (((1,), (0,)), ((), ())),
            preferred_element_type=jnp.float32)
        y = y + b2_ref[e]
        w = cw[:, e:e + 1]
        acc = acc + w * y
    out_ref[...] = acc


@functools.partial(jax.jit)
def kernel(x, gate_W, gate_b, W1, b1, W2, b2):
    gw_pad = jnp.zeros((D, EP), jnp.float32).at[:, :E].set(gate_W)
    gb_pad = jnp.zeros((8, EP), jnp.float32).at[:, :E].set(gate_b[None, :])
    w1_16 = W1.astype(jnp.bfloat16)
    w2_16 = W2.astype(jnp.bfloat16)
    b1r = b1.reshape(E, 1, D)
    b2r = b2.reshape(E, 1, C)

    out = pl.pallas_call(
        _moe_dense_kernel,
        grid=(T // BT,),
        in_specs=[
            pl.BlockSpec((BT, D), lambda t: (t, 0)),
            pl.BlockSpec((D, EP), lambda t: (0, 0)),
            pl.BlockSpec((8, EP), lambda t: (0, 0)),
            pl.BlockSpec((E, D, D), lambda t: (0, 0, 0)),
            pl.BlockSpec((E, 1, D), lambda t: (0, 0, 0)),
            pl.BlockSpec((E, D, C), lambda t: (0, 0, 0)),
            pl.BlockSpec((E, 1, C), lambda t: (0, 0, 0)),
        ],
        out_specs=pl.BlockSpec((BT, C), lambda t: (t, 0)),
        out_shape=jax.ShapeDtypeStruct((T, C), jnp.float32),
    )(x, gw_pad, gb_pad, w1_16, b1r, w2_16, b2r)
    return out
